# baseline (speedup 1.0000x reference)
import functools

import jax
import jax.numpy as jnp
from jax import lax
from jax.experimental import pallas as pl
from jax.experimental.pallas import tpu as pltpu
from jax.experimental.pallas import tpu_sc as plsc

_TOKENS = 8192
_D_MODEL = 2048
_NUM_EXPERTS = 64
_TOP_K = 8
_NW = 32
_L = 16
_BLK_T = 2048
_NCHUNK = 2
_CTOK = _TOKENS // _NCHUNK
_TPW = _CTOK // _NW
_WVOL = _NUM_EXPERTS * _TPW
_WIDX = _TOP_K * _TPW


def _logits_block(x_ref, w_ref, b_ref, noise_ref, out_ref):
    x = x_ref[...]
    w = w_ref[...]
    l = jax.lax.dot_general(w, x, (((1,), (1,)), ((), ())),
                            preferred_element_type=jnp.float32)
    l = l + b_ref[...] + noise_ref[...]
    for k in range(_BLK_T // _TPW):
        out_ref[k] = l[:, k * _TPW:(k + 1) * _TPW]


def _compute_logits(x, W, b2, noise, chunk):
    nblk = _CTOK // _BLK_T
    off = chunk * nblk
    return pl.pallas_call(
        _logits_block,
        grid=(nblk,),
        in_specs=[
            pl.BlockSpec((_BLK_T, _D_MODEL), lambda i: (i + off, 0)),
            pl.BlockSpec((_NUM_EXPERTS, _D_MODEL), lambda i: (0, 0)),
            pl.BlockSpec((_NUM_EXPERTS, 1), lambda i: (0, 0)),
            pl.BlockSpec((_NUM_EXPERTS, _BLK_T), lambda i: (0, i + off)),
        ],
        out_specs=pl.BlockSpec((_BLK_T // _TPW, _NUM_EXPERTS, _TPW),
                               lambda i: (i, 0, 0)),
        out_shape=jax.ShapeDtypeStruct((_NW, _NUM_EXPERTS, _TPW), jnp.float32),
    )(x, W, b2, noise)


_sc_mesh = plsc.VectorSubcoreMesh(core_axis_name="c", subcore_axis_name="s")


@functools.partial(
    pl.kernel,
    mesh=_sc_mesh,
    out_type=[
        jax.ShapeDtypeStruct((_NW, _NUM_EXPERTS, _TPW), jnp.float32),
        jax.ShapeDtypeStruct((_NW, _TOP_K, _TPW), jnp.int32),
    ],
    scratch_types=[
        pltpu.VMEM((_NUM_EXPERTS, _TPW), jnp.float32),
        pltpu.VMEM((_NUM_EXPERTS, _TPW), jnp.float32),
        pltpu.VMEM((_TOP_K, _TPW), jnp.int32),
    ],
    compiler_params=pltpu.CompilerParams(needs_layout_passes=False),
)
def _sc_topk_softmax(logits_hbm, zeros_hbm, out_hbm, idx_hbm, l_v, o_v, i_v):
    w = lax.axis_index("s") * 2 + lax.axis_index("c")
    pltpu.sync_copy(logits_hbm.at[w], l_v)
    pltpu.sync_copy(zeros_hbm, o_v)
    lanes = lax.iota(jnp.int32, _L)

    _GPB = 1

    def group_body(g, _):
        col = pl.multiple_of(g * (_L * _GPB), _L)

        def expert_body(e, carry):
            ei0 = jnp.zeros((_L,), jnp.int32) + e
            out = []
            for s in range(_GPB):
                ts = carry[s * 2 * _TOP_K: s * 2 * _TOP_K + _TOP_K]
                iz = carry[s * 2 * _TOP_K + _TOP_K: (s + 1) * 2 * _TOP_K]
                v = l_v[e, pl.ds(col + s * _L, _L)]
                ei = ei0
                nts, nis = [], []
                for j in range(_TOP_K):
                    gt = v > ts[j]
                    nts.append(jnp.where(gt, v, ts[j]))
                    nis.append(jnp.where(gt, ei, iz[j]))
                    v = jnp.where(gt, ts[j], v)
                    ei = jnp.where(gt, iz[j], ei)
                out += nts + nis
            return tuple(out)

        init = tuple(
            jnp.full((_L,), -jnp.inf, jnp.float32) if (k // _TOP_K) % 2 == 0
            else jnp.zeros((_L,), jnp.int32)
            for k in range(2 * _TOP_K * _GPB))
        carry = lax.fori_loop(0, _NUM_EXPERTS, expert_body, init)
        for s in range(_GPB):
            ts = carry[s * 2 * _TOP_K: s * 2 * _TOP_K + _TOP_K]
            iz = carry[s * 2 * _TOP_K + _TOP_K: (s + 1) * 2 * _TOP_K]
            m0 = ts[0]
            exps = [jnp.exp(ts[j] - m0) for j in range(_TOP_K)]
            z = exps[0]
            for j in range(1, _TOP_K):
                z = z + exps[j]
            toks = col + s * _L + lanes
            for j in range(_TOP_K):
                plsc.store_scatter(o_v, [iz[j], toks], exps[j] / z)
                i_v[j, pl.ds(col + s * _L, _L)] = iz[j]
        return 0

    lax.fori_loop(0, _TPW // (_L * _GPB), group_body, 0)
    pltpu.sync_copy(o_v, out_hbm.at[w])
    pltpu.sync_copy(i_v, idx_hbm.at[w])


def kernel(x, W, b, noise):
    noise_t = noise.T
    b2 = b.reshape(_NUM_EXPERTS, 1)
    zeros = jnp.zeros((_NUM_EXPERTS, _TPW), jnp.float32)
    outs, idxs = [], []
    for c in range(_NCHUNK):
        logits3 = _compute_logits(x, W, b2, noise_t, c)
        out3, idx3 = _sc_topk_softmax(logits3, zeros)
        outs.append(out3.transpose(0, 2, 1).reshape(_CTOK, _NUM_EXPERTS))
        idxs.append(idx3.transpose(0, 2, 1).reshape(_CTOK, _TOP_K))
    return (jnp.concatenate(outs, axis=0), jnp.concatenate(idxs, axis=0))


# speedup vs baseline: 1.0229x; 1.0229x over previous
import functools

import jax
import jax.numpy as jnp
from jax import lax
from jax.experimental import pallas as pl
from jax.experimental.pallas import tpu as pltpu
from jax.experimental.pallas import tpu_sc as plsc

_TOKENS = 8192
_D_MODEL = 2048
_NUM_EXPERTS = 64
_TOP_K = 8
_NW = 32
_L = 16
_BLK_T = 1024
_NCHUNK = 2
_CTOK = _TOKENS // _NCHUNK
_TPW = _CTOK // _NW
_WVOL = _NUM_EXPERTS * _TPW
_WIDX = _TOP_K * _TPW


def _logits_block(x_ref, w_ref, b_ref, noise_ref, out_ref):
    x = x_ref[...]
    w = w_ref[...]
    l = jax.lax.dot_general(w, x, (((1,), (1,)), ((), ())),
                            preferred_element_type=jnp.float32)
    l = l + b_ref[...] + noise_ref[...]
    for k in range(_BLK_T // _TPW):
        out_ref[k] = l[:, k * _TPW:(k + 1) * _TPW]


def _compute_logits(x, W, b2, noise, chunk):
    nblk = _CTOK // _BLK_T
    off = chunk * nblk
    return pl.pallas_call(
        _logits_block,
        grid=(nblk,),
        in_specs=[
            pl.BlockSpec((_BLK_T, _D_MODEL), lambda i: (i + off, 0)),
            pl.BlockSpec((_NUM_EXPERTS, _D_MODEL), lambda i: (0, 0)),
            pl.BlockSpec((_NUM_EXPERTS, 1), lambda i: (0, 0)),
            pl.BlockSpec((_NUM_EXPERTS, _BLK_T), lambda i: (0, i + off)),
        ],
        out_specs=pl.BlockSpec((_BLK_T // _TPW, _NUM_EXPERTS, _TPW),
                               lambda i: (i, 0, 0)),
        out_shape=jax.ShapeDtypeStruct((_NW, _NUM_EXPERTS, _TPW), jnp.float32),
    )(x, W, b2, noise)


_sc_mesh = plsc.VectorSubcoreMesh(core_axis_name="c", subcore_axis_name="s")


@functools.partial(
    pl.kernel,
    mesh=_sc_mesh,
    out_type=[
        jax.ShapeDtypeStruct((_NW, _NUM_EXPERTS, _TPW), jnp.float32),
        jax.ShapeDtypeStruct((_NW, _TOP_K, _TPW), jnp.int32),
    ],
    scratch_types=[
        pltpu.VMEM((_NUM_EXPERTS, _TPW), jnp.float32),
        pltpu.VMEM((_NUM_EXPERTS, _TPW), jnp.float32),
        pltpu.VMEM((_TOP_K, _TPW), jnp.int32),
    ],
    compiler_params=pltpu.CompilerParams(needs_layout_passes=False),
)
def _sc_topk_softmax(logits_hbm, zeros_hbm, out_hbm, idx_hbm, l_v, o_v, i_v):
    w = lax.axis_index("s") * 2 + lax.axis_index("c")
    pltpu.sync_copy(logits_hbm.at[w], l_v)
    pltpu.sync_copy(zeros_hbm, o_v)
    lanes = lax.iota(jnp.int32, _L)

    _GPB = 1

    def group_body(g, _):
        col = pl.multiple_of(g * (_L * _GPB), _L)

        def expert_body(e, carry):
            ei0 = jnp.zeros((_L,), jnp.int32) + e
            out = []
            for s in range(_GPB):
                ts = carry[s * 2 * _TOP_K: s * 2 * _TOP_K + _TOP_K]
                iz = carry[s * 2 * _TOP_K + _TOP_K: (s + 1) * 2 * _TOP_K]
                v = l_v[e, pl.ds(col + s * _L, _L)]
                ei = ei0
                nts, nis = [], []
                for j in range(_TOP_K):
                    gt = v > ts[j]
                    nts.append(jnp.where(gt, v, ts[j]))
                    nis.append(jnp.where(gt, ei, iz[j]))
                    v = jnp.where(gt, ts[j], v)
                    ei = jnp.where(gt, iz[j], ei)
                out += nts + nis
            return tuple(out)

        init = tuple(
            jnp.full((_L,), -jnp.inf, jnp.float32) if (k // _TOP_K) % 2 == 0
            else jnp.zeros((_L,), jnp.int32)
            for k in range(2 * _TOP_K * _GPB))
        carry = lax.fori_loop(0, _NUM_EXPERTS, expert_body, init)
        for s in range(_GPB):
            ts = carry[s * 2 * _TOP_K: s * 2 * _TOP_K + _TOP_K]
            iz = carry[s * 2 * _TOP_K + _TOP_K: (s + 1) * 2 * _TOP_K]
            m0 = ts[0]
            exps = [jnp.exp(ts[j] - m0) for j in range(_TOP_K)]
            z = exps[0]
            for j in range(1, _TOP_K):
                z = z + exps[j]
            toks = col + s * _L + lanes
            for j in range(_TOP_K):
                plsc.store_scatter(o_v, [iz[j], toks], exps[j] / z)
                i_v[j, pl.ds(col + s * _L, _L)] = iz[j]
        return 0

    lax.fori_loop(0, _TPW // (_L * _GPB), group_body, 0)
    pltpu.sync_copy(o_v, out_hbm.at[w])
    pltpu.sync_copy(i_v, idx_hbm.at[w])


def kernel(x, W, b, noise):
    noise_t = noise.T
    b2 = b.reshape(_NUM_EXPERTS, 1)
    zeros = jnp.zeros((_NUM_EXPERTS, _TPW), jnp.float32)
    outs, idxs = [], []
    for c in range(_NCHUNK):
        logits3 = _compute_logits(x, W, b2, noise_t, c)
        out3, idx3 = _sc_topk_softmax(logits3, zeros)
        outs.append(out3.transpose(0, 2, 1).reshape(_CTOK, _NUM_EXPERTS))
        idxs.append(idx3.transpose(0, 2, 1).reshape(_CTOK, _TOP_K))
    return (jnp.concatenate(outs, axis=0), jnp.concatenate(idxs, axis=0))
